# baseline (speedup 1.0000x reference)
import functools

import jax
import jax.numpy as jnp
from jax import lax
from jax.experimental import pallas as pl
from jax.experimental.pallas import tpu as pltpu
from jax.experimental.pallas import tpu_sc as plsc


def _fold_body(emb_ref, w_ref, b_ref, out_ref):
    t = lax.dot_general(
        emb_ref[...], w_ref[...],
        (((1,), (1,)), ((), ())),
        preferred_element_type=jnp.float32,
        precision=lax.Precision.HIGHEST,
    )
    out_ref[...] = t + b_ref[...]


def _fold_table(emb, W, b2d):
    V, _ = emb.shape
    C = W.shape[0]
    return pl.pallas_call(
        _fold_body,
        out_shape=jax.ShapeDtypeStruct((V, C), jnp.float32),
    )(emb, W, b2d)


def _gather_rows(table, idx_flat):
    V, C = table.shape
    B = idx_flat.shape[0]
    info = plsc.get_sparse_core_info()
    nc, ns = info.num_cores, info.num_subcores
    nw = nc * ns
    b_per_w = B // nw
    G = 128
    NG = 10
    CH = G * NG
    n_ch = b_per_w // CH
    RP = CH * C // 128
    n_g_w = b_per_w // G
    mesh = plsc.VectorSubcoreMesh(core_axis_name="c", subcore_axis_name="s")

    @functools.partial(
        pl.kernel,
        mesh=mesh,
        out_type=jax.ShapeDtypeStruct((B * C // 128, 128), jnp.float32),
        scratch_types=[
            pltpu.VMEM((n_g_w, G), jnp.int32),
            pltpu.VMEM((CH, C), jnp.float32),
            pltpu.VMEM((RP, 128), jnp.float32),
            pltpu.SemaphoreType.DMA,
        ],
        compiler_params=pltpu.CompilerParams(use_tc_tiling_on_sc=False),
    )
    def k(table_hbm, idx_hbm, out_hbm, idx_v, rows_v, pack_v, sem):
        wid = lax.axis_index("s") * nc + lax.axis_index("c")
        base_g = wid * n_g_w
        base_p = wid * (b_per_w * C // 128)
        pltpu.sync_copy(idx_hbm.at[pl.ds(base_g, n_g_w)], idx_v)

        def body(i, carry):
            for j in range(NG):
                pltpu.async_copy(
                    table_hbm.at[idx_v.at[i * NG + j]],
                    rows_v.at[pl.ds(j * G, G)],
                    sem,
                )
            for j in range(NG):
                pltpu.make_async_copy(
                    table_hbm.at[idx_v.at[i * NG + j]],
                    rows_v.at[pl.ds(j * G, G)],
                    sem,
                ).wait()

            def pack(p, c2):
                for kk in range(4):
                    for h in range(2):
                        pack_v[p, pl.ds(kk * 32 + h * 16, 16)] = (
                            rows_v[4 * p + kk, pl.ds(h * 16, 16)]
                        )
                return c2

            lax.fori_loop(0, RP, pack, 0)
            pltpu.sync_copy(pack_v, out_hbm.at[pl.ds(base_p + i * RP, RP)])
            return carry

        lax.fori_loop(0, n_ch, body, 0)

    return k(table, idx_flat.reshape(B // G, G)).reshape(B, C)


def _round_up(n, m):
    return (n + m - 1) // m * m


def kernel(x, emb, W, b):
    C = W.shape[0]
    Cp = _round_up(C, 32)
    Wp = jnp.pad(W.astype(jnp.float32), ((0, Cp - C), (0, 0)))
    bp = jnp.pad(b.astype(jnp.float32), (0, Cp - C)).reshape(1, Cp)
    tab = _fold_table(emb, Wp, bp)
    idx = x.reshape(-1).astype(jnp.int32)
    out = _gather_rows(tab, idx)
    return out[:, :C].reshape(x.shape[0], x.shape[1], C)


# speedup vs baseline: 2.2535x; 2.2535x over previous
import functools

import jax
import jax.numpy as jnp
from jax import lax
from jax.experimental import pallas as pl
from jax.experimental.pallas import tpu as pltpu
from jax.experimental.pallas import tpu_sc as plsc


def _fold_body(emb_ref, w_ref, b_ref, out_ref):
    t = lax.dot_general(
        emb_ref[...], w_ref[...],
        (((1,), (1,)), ((), ())),
        preferred_element_type=jnp.float32,
        precision=lax.Precision.HIGHEST,
    )
    out_ref[...] = t + b_ref[...]


def _fold_table(emb, W, b2d):
    V, _ = emb.shape
    Cp = W.shape[0]
    return pl.pallas_call(
        _fold_body,
        out_shape=jax.ShapeDtypeStruct((V, Cp), jnp.float32),
    )(emb, W, b2d)


def _gather_transposed(table, idx2d, NB, L, C):
    V, Cp = table.shape
    info = plsc.get_sparse_core_info()
    nc, ns = info.num_cores, info.num_subcores
    nw = nc * ns
    Lt = (L + 7) // 8
    n_rows = C * Lt * 8 * nw
    mesh = plsc.VectorSubcoreMesh(core_axis_name="c", subcore_axis_name="s")

    iota16 = lambda: lax.iota(jnp.int32, 16)

    @functools.partial(
        pl.kernel,
        mesh=mesh,
        out_type=jax.ShapeDtypeStruct((n_rows, 128), jnp.float32),
        scratch_types=[
            pltpu.VMEM((L, 128), jnp.int32),
            pltpu.VMEM((128, Cp), jnp.float32),
            pltpu.VMEM((128, Cp), jnp.float32),
            pltpu.VMEM((C, 8, 128), jnp.float32),
            pltpu.SemaphoreType.DMA,
            pltpu.SemaphoreType.DMA,
            pltpu.SemaphoreType.DMA,
            pltpu.SemaphoreType.DMA,
        ],
        compiler_params=pltpu.CompilerParams(
            use_tc_tiling_on_sc=False, needs_layout_passes=False),
    )
    def k(table_hbm, idx_hbm, out_hbm, idx_v, gba, gbb, pack_v,
          sem_i, sem_a, sem_b, sem_o):
        wid = lax.axis_index("s") * nc + lax.axis_index("c")

        for j in range(L):
            pltpu.async_copy(idx_hbm.at[j * 32 + wid], idx_v.at[j], sem_i)
        for j in range(L):
            pltpu.make_async_copy(
                idx_hbm.at[j * 32 + wid], idx_v.at[j], sem_i).wait()

        def fire(j, gb, sem):
            pltpu.async_copy(table_hbm.at[idx_v.at[j]], gb, sem)

        def wait(j, gb, sem):
            pltpu.make_async_copy(table_hbm.at[idx_v.at[j]], gb, sem).wait()

        def extract(gb, jl):
            for h in range(8):
                rows = iota16() + (16 * h)
                for c in range(C):
                    cols = jnp.full((16,), c, jnp.int32)
                    v = plsc.load_gather(gb, [rows, cols])
                    pack_v[c, jl, pl.ds(h * 16, 16)] = v

        def emit(jt):
            for c in range(C):
                r0 = (c * Lt + jt) * (8 * nw) + wid * 8
                pltpu.async_copy(pack_v.at[c], out_hbm.at[pl.ds(r0, 8)], sem_o)
            for c in range(C):
                r0 = (c * Lt + jt) * (8 * nw) + wid * 8
                pltpu.make_async_copy(
                    pack_v.at[c], out_hbm.at[pl.ds(r0, 8)], sem_o).wait()

        fire(0, gba, sem_a)
        fire(1, gbb, sem_b)

        def body(j2, carry):
            j = j2 * 2
            jt = j // 8

            wait(j, gba, sem_a)
            extract(gba, j - jt * 8)

            @pl.when(j + 2 < L)
            def _():
                fire(j + 2, gba, sem_a)

            wait(j + 1, gbb, sem_b)
            extract(gbb, j + 1 - jt * 8)

            @pl.when(j + 3 < L)
            def _():
                fire(j + 3, gbb, sem_b)

            @pl.when(jnp.logical_or(j + 1 - jt * 8 == 7, j + 1 == L - 1))
            def _():
                emit(jt)
            return carry

        lax.fori_loop(0, L // 2, body, 0)

    return k(table, idx2d)


def kernel(x, emb, W, b):
    NB, L = x.shape
    C = W.shape[0]
    Cp = 32
    Wp = jnp.pad(W.astype(jnp.float32), ((0, Cp - C), (0, 0)))
    bp = jnp.pad(b.astype(jnp.float32), (0, Cp - C)).reshape(1, Cp)
    tab = _fold_table(emb, Wp, bp)

    idx2d = x.T.astype(jnp.int32).reshape(L * NB // 128, 128)
    raw = _gather_transposed(tab, idx2d, NB, L, C)

    Lt = (L + 7) // 8
    r = raw.reshape(C, Lt, NB // 128, 8, 128)
    t = r.transpose(2, 4, 1, 3, 0)
    f = t.reshape(NB, Lt * 8, C)
    return f[:, :L, :]


# speedup vs baseline: 2.4606x; 1.0919x over previous
import functools

import jax
import jax.numpy as jnp
from jax import lax
from jax.experimental import pallas as pl
from jax.experimental.pallas import tpu as pltpu
from jax.experimental.pallas import tpu_sc as plsc


def _fold_body(emb_ref, w_ref, b_ref, out_ref):
    t = lax.dot_general(
        emb_ref[...], w_ref[...],
        (((1,), (1,)), ((), ())),
        preferred_element_type=jnp.float32,
        precision=lax.Precision.HIGHEST,
    )
    out_ref[...] = t + b_ref[...]


def _fold_table(emb, W, b2d):
    V, _ = emb.shape
    Cp = W.shape[0]
    return pl.pallas_call(
        _fold_body,
        out_shape=jax.ShapeDtypeStruct((V, Cp), jnp.float32),
    )(emb, W, b2d)


def _gather_transposed(table1d, idxw, NB, L, C):
    VW = table1d.shape[0]
    info = plsc.get_sparse_core_info()
    nc, ns = info.num_cores, info.num_subcores
    nw = nc * ns
    Lt = (L + 7) // 8
    n_rows = C * Lt * 8 * nw
    mesh = plsc.VectorSubcoreMesh(core_axis_name="c", subcore_axis_name="s")

    @functools.partial(
        pl.kernel,
        mesh=mesh,
        out_type=jax.ShapeDtypeStruct((n_rows, 128), jnp.float32),
        scratch_types=[
            pltpu.VMEM((VW,), jnp.float32),
            pltpu.VMEM((L, 128), jnp.int32),
            pltpu.VMEM((2, C, 8, 128), jnp.float32),
            pltpu.SemaphoreType.DMA,
            pltpu.SemaphoreType.DMA,
        ],
        compiler_params=pltpu.CompilerParams(
            use_tc_tiling_on_sc=False, needs_layout_passes=False),
    )
    def k(table_hbm, idx_hbm, out_hbm, tab_v, idx_v, pack_v, sem_e, sem_o):
        wid = lax.axis_index("s") * nc + lax.axis_index("c")
        pltpu.sync_copy(table_hbm, tab_v)
        pltpu.sync_copy(idx_hbm.at[pl.ds(wid * L, L)], idx_v)

        def tile_rows(c, jt):
            return (c * Lt + jt) * (8 * nw) + wid * 8

        def fire(par, jt):
            sem = sem_e if par == 0 else sem_o
            for c in range(C):
                pltpu.async_copy(
                    pack_v.at[par, c],
                    out_hbm.at[pl.ds(tile_rows(c, jt), 8)], sem)

        def drain(par):
            sem = sem_e if par == 0 else sem_o
            for c in range(C):
                pltpu.make_async_copy(
                    pack_v.at[par, c], out_hbm.at[pl.ds(wid * 8, 8)],
                    sem).wait()

        def body(j, carry):
            jt = j // 8
            jl = j - jt * 8
            par = jt & 1

            @pl.when(jnp.logical_and(jl == 0, jt >= 2))
            def _():
                @pl.when(par == 0)
                def _():
                    drain(0)
                @pl.when(par == 1)
                def _():
                    drain(1)

            for h in range(8):
                idx16 = idx_v[j, pl.ds(h * 16, 16)]
                rows32 = idx16 * 32
                for c in range(C):
                    v = plsc.load_gather(tab_v, [rows32 + c])
                    pack_v[par, c, jl, pl.ds(h * 16, 16)] = v

            @pl.when(jnp.logical_or(jl == 7, j == L - 1))
            def _():
                @pl.when(par == 0)
                def _():
                    fire(0, jt)
                @pl.when(par == 1)
                def _():
                    fire(1, jt)
            return carry

        lax.fori_loop(0, L, body, 0)
        drain(Lt % 2)
        drain((Lt - 1) % 2)

    return k(table1d, idxw)


def kernel(x, emb, W, b):
    NB, L = x.shape
    C = W.shape[0]
    Cp = 32
    Wp = jnp.pad(W.astype(jnp.float32), ((0, Cp - C), (0, 0)))
    bp = jnp.pad(b.astype(jnp.float32), (0, Cp - C)).reshape(1, Cp)
    tab = _fold_table(emb, Wp, bp)

    idxw = (x.T.astype(jnp.int32)
            .reshape(L, NB // 128, 128)
            .transpose(1, 0, 2)
            .reshape(NB // 128 * L, 128))
    raw = _gather_transposed(tab.reshape(-1), idxw, NB, L, C)

    Lt = (L + 7) // 8
    r = raw.reshape(C, Lt, NB // 128, 8, 128)
    t = r.transpose(2, 4, 1, 3, 0)
    f = t.reshape(NB, Lt * 8, C)
    return f[:, :L, :]
